# baseline (device time: 92274 ns/iter reference)
import jax
import jax.numpy as jnp
from jax import lax
from jax.experimental import pallas as pl
from jax.experimental.pallas import tpu as pltpu

N_DEV = 4
N_SUB = 2


def _gelu(y):
    c = 0.7978845608028654
    return 0.5 * y * (1.0 + jnp.tanh(c * (y + 0.044715 * y * y * y)))


def kernel(x, w_mat):
    m, k = x.shape
    _, n = w_mat.shape
    m_per = m // N_DEV
    m_sub = m_per // N_SUB
    n_half = n // 2

    def body(x_ref, w_ref, out_ref, xb_ref, wb_ref, res_ref, cw_ref, ccw_ref,
             cw_send, cw_recv, ccw_send, ccw_recv, out_sems):
        my = lax.axis_index("i")
        left = lax.rem(my + N_DEV - 1, N_DEV)
        right = lax.rem(my + 1, N_DEV)

        barrier_sem = pltpu.get_barrier_semaphore()
        for nbr in (left, right):
            pl.semaphore_signal(
                barrier_sem, inc=1,
                device_id=(nbr,), device_id_type=pl.DeviceIdType.MESH,
            )
        xb_ref[...] = x_ref[...].astype(jnp.bfloat16)
        wb_ref[...] = w_ref[...].astype(jnp.bfloat16)
        pl.semaphore_wait(barrier_sem, 2)

        def part(chunk, s, lo):
            xs = xb_ref[pl.ds(chunk * m_per + s * m_sub, m_sub), :]
            ws = wb_ref[:, 0:n_half] if lo else wb_ref[:, n_half:n]
            return jnp.dot(xs, ws, preferred_element_type=jnp.float32)

        def mk(h, s, ref, s_sems, r_sems, dev):
            src = N_DEV - 1 if h == 0 else h - 1
            return pltpu.make_async_remote_copy(
                src_ref=ref.at[src, s], dst_ref=ref.at[h, s],
                send_sem=s_sems.at[h, s], recv_sem=r_sems.at[h, s],
                device_id=(dev,), device_id_type=pl.DeviceIdType.MESH,
            )

        c_cw0 = lax.rem(my + N_DEV - 1, N_DEV)
        c_ccw0 = lax.rem(my + 1, N_DEV)
        rd = {}
        for s in range(N_SUB):
            cw_ref[N_DEV - 1, s, :, :] = part(c_cw0, s, True).astype(jnp.bfloat16)
            rd["cw", 0, s] = mk(0, s, cw_ref, cw_send, cw_recv, right)
            rd["cw", 0, s].start()
            ccw_ref[N_DEV - 1, s, :, :] = part(c_ccw0, s, False).astype(jnp.bfloat16)
            rd["ccw", 0, s] = mk(0, s, ccw_ref, ccw_send, ccw_recv, left)
            rd["ccw", 0, s].start()

        out_dmas = []
        for h in range(N_DEV - 1):
            c_cw = lax.rem(my + 3 * N_DEV - 2 - h, N_DEV)
            c_ccw = lax.rem(my + 2 + h, N_DEV)
            for s in range(N_SUB):
                a_cw = part(c_cw, s, True)
                a_ccw = part(c_ccw, s, False)
                for i, (d, a, ref, s_sems, r_sems, dev) in enumerate((
                    ("cw", a_cw, cw_ref, cw_send, cw_recv, right),
                    ("ccw", a_ccw, ccw_ref, ccw_send, ccw_recv, left),
                )):
                    rd[d, h, s].wait_recv()
                    if h < N_DEV - 2:
                        ref[h, s, :, :] = (
                            ref[h, s, :, :].astype(jnp.float32) + a
                        ).astype(jnp.bfloat16)
                        rd[d, h + 1, s] = mk(h + 1, s, ref, s_sems, r_sems, dev)
                        rd[d, h + 1, s].start()
                    else:
                        col0 = 0 if d == "cw" else n_half
                        res_ref[s, i, :, :] = _gelu(
                            ref[h, s, :, :].astype(jnp.float32) + a)
                        dma = pltpu.make_async_copy(
                            res_ref.at[s, i],
                            out_ref.at[pl.ds(s * m_sub, m_sub),
                                       pl.ds(col0, n_half)],
                            out_sems.at[s, i],
                        )
                        dma.start()
                        out_dmas.append(dma)
                    rd[d, h, s].wait_send()
        for dma in out_dmas:
            dma.wait()

    return pl.pallas_call(
        body,
        out_shape=jax.ShapeDtypeStruct((m_per, n), jnp.float32),
        in_specs=[
            pl.BlockSpec(memory_space=pltpu.VMEM),
            pl.BlockSpec(memory_space=pltpu.VMEM),
        ],
        out_specs=pl.BlockSpec(memory_space=pl.ANY),
        scratch_shapes=[
            pltpu.VMEM((m, k), jnp.bfloat16),
            pltpu.VMEM((k, n), jnp.bfloat16),
            pltpu.VMEM((N_SUB, 2, m_sub, n_half), jnp.float32),
            pltpu.VMEM((N_DEV, N_SUB, m_sub, n_half), jnp.bfloat16),
            pltpu.VMEM((N_DEV, N_SUB, m_sub, n_half), jnp.bfloat16),
            pltpu.SemaphoreType.DMA((N_DEV - 1, N_SUB)),
            pltpu.SemaphoreType.DMA((N_DEV - 1, N_SUB)),
            pltpu.SemaphoreType.DMA((N_DEV - 1, N_SUB)),
            pltpu.SemaphoreType.DMA((N_DEV - 1, N_SUB)),
            pltpu.SemaphoreType.DMA((N_SUB, 2)),
        ],
        compiler_params=pltpu.CompilerParams(
            collective_id=0, vmem_limit_bytes=110 * 1024 * 1024,
        ),
    )(x, w_mat)
